# baseline (speedup 1.0000x reference)
import jax
import jax.numpy as jnp
from jax import lax
from jax.experimental import pallas as pl
from jax.experimental.pallas import tpu as pltpu
from jax.experimental.pallas import tpu_sc as plsc

_RATE = 2
_B, _T, _D = 16, 4096, 256
_TO = _T // _RATE
_C = 64
_NWORK = 32
_NCHG = (_B * _TO) // _C
_SLOTS = _NCHG // _NWORK
_JPB = _TO // _C
_L = 16
_NBUF = 5
_ZR = 64
_PERM = 173


def _sc_body(padded_hbm, lengths_hbm, out_hbm, nl_hbm,
             idx0, idx1, idx2, idx3, idx4,
             gbuf0, gbuf1, gbuf2, gbuf3, gbuf4, zbuf, lens_v, nl_v,
             gsem0, gsem1, gsem2, gsem3, gsem4,
             osem0, osem1, osem2, osem3, osem4, zsem):
    idxb = (idx0, idx1, idx2, idx3, idx4)
    gbuf = (gbuf0, gbuf1, gbuf2, gbuf3, gbuf4)
    gsem = (gsem0, gsem1, gsem2, gsem3, gsem4)
    osem = (osem0, osem1, osem2, osem3, osem4)

    wid = lax.axis_index("s") * 2 + lax.axis_index("c")

    pltpu.sync_copy(lengths_hbm, lens_v)
    lane = lax.iota(jnp.int32, _L)
    lens = lens_v[...]

    @pl.when(wid == 0)
    def _():
        nl_v[...] = lens // _RATE
        pltpu.sync_copy(nl_v, nl_hbm)

    zeros16 = jnp.zeros((_L,), jnp.float32)

    @pl.loop(0, _ZR)
    def _(r):
        for j in range(_D // _L):
            zbuf[r, pl.ds(j * _L, _L)] = zeros16

    kg, bb, vk, orow, row0 = [], [], [], [], []
    for i in range(_SLOTS):
        k = (_PERM * (_SLOTS * wid + i)) % _NCHG
        b = k // _JPB
        j = k % _JPB
        nl = jnp.sum(jnp.where(lane == b, lens, 0)) // _RATE
        kg.append(k)
        bb.append(b)
        vk.append(jnp.clip(nl - j * _C, 0, _C))
        orow.append(k * _C)
        row0.append(b * _T + 2 * (j * _C) + 1)

    for i in range(_SLOTS):
        @pl.when(vk[i] == 0)
        def _(i=i):
            pltpu.async_copy(zbuf, out_hbm.at[pl.ds(orow[i], _C)], zsem)

    def gather_desc(i):
        s = i % _NBUF
        return pltpu.make_async_copy(
            padded_hbm.at[idxb[s]], gbuf[s], gsem[s])

    def out_desc(i):
        s = i % _NBUF
        return pltpu.make_async_copy(
            gbuf[s], out_hbm.at[pl.ds(orow[i], _C)], osem[s])

    def start_gather(i):
        s = i % _NBUF
        for j in range(_C // _L):
            idxb[s][pl.ds(j * _L, _L)] = row0[i] + 2 * (j * _L + lane)
        gather_desc(i).start()

    def finish_gather_start_out(i):
        gather_desc(i).wait()
        s = i % _NBUF

        @pl.when(vk[i] < _C)
        def _():
            @pl.loop(vk[i], _C)
            def _(r):
                for j in range(_D // _L):
                    gbuf[s][r, pl.ds(j * _L, _L)] = zeros16

        out_desc(i).start()

    for i in range(_SLOTS):
        if i >= _NBUF:
            @pl.when(vk[i - _NBUF] > 0)
            def _(i=i):
                out_desc(i - _NBUF).wait()

        @pl.when(vk[i] > 0)
        def _(i=i):
            start_gather(i)
        if i >= 1:
            @pl.when(vk[i - 1] > 0)
            def _(i=i):
                finish_gather_start_out(i - 1)

    @pl.when(vk[_SLOTS - 1] > 0)
    def _():
        finish_gather_start_out(_SLOTS - 1)

    for i in range(max(_SLOTS - _NBUF, 0), _SLOTS):
        @pl.when(vk[i] > 0)
        def _(i=i):
            out_desc(i).wait()

    for i in range(_SLOTS):
        @pl.when(vk[i] == 0)
        def _(i=i):
            pltpu.make_async_copy(
                zbuf, out_hbm.at[pl.ds(orow[i], _C)], zsem).wait()


def kernel(padded, lengths):
    padded2d = padded.reshape(_B * _T, _D)
    mesh = plsc.VectorSubcoreMesh(core_axis_name="c", subcore_axis_name="s")
    out2d, nl = pl.kernel(
        _sc_body,
        out_type=(
            jax.ShapeDtypeStruct((_B * _TO, _D), jnp.float32),
            jax.ShapeDtypeStruct((_B,), jnp.int32),
        ),
        mesh=mesh,
        compiler_params=pltpu.CompilerParams(needs_layout_passes=False),
        scratch_types=(
            *([pltpu.VMEM((_C,), jnp.int32)] * _NBUF),
            *([pltpu.VMEM((_C, _D), jnp.float32)] * _NBUF),
            pltpu.VMEM((_ZR, _D), jnp.float32),
            pltpu.VMEM((_L,), jnp.int32),
            pltpu.VMEM((_L,), jnp.int32),
            *([pltpu.SemaphoreType.DMA] * _NBUF),
            *([pltpu.SemaphoreType.DMA] * _NBUF),
            pltpu.SemaphoreType.DMA,
        ),
    )(padded2d, lengths)
    return out2d.reshape(_B, _TO, _D), nl


# speedup vs baseline: 1.0511x; 1.0511x over previous
import jax
import jax.numpy as jnp
from jax import lax
from jax.experimental import pallas as pl
from jax.experimental.pallas import tpu as pltpu
from jax.experimental.pallas import tpu_sc as plsc

_RATE = 2
_B, _T, _D = 16, 4096, 256
_TO = _T // _RATE
_C = 128
_NWORK = 32
_NCHG = (_B * _TO) // _C
_SLOTS = _NCHG // _NWORK
_JPB = _TO // _C
_L = 16
_NBUF = 3
_ZR = 64
_PERM = 173


def _sc_body(padded_hbm, lengths_hbm, out_hbm, nl_hbm,
             idx0, idx1, idx2, gbuf0, gbuf1, gbuf2, zbuf, lens_v, nl_v,
             gsem0, gsem1, gsem2, osem0, osem1, osem2, zsem):
    idxb = (idx0, idx1, idx2)
    gbuf = (gbuf0, gbuf1, gbuf2)
    gsem = (gsem0, gsem1, gsem2)
    osem = (osem0, osem1, osem2)

    wid = lax.axis_index("s") * 2 + lax.axis_index("c")

    pltpu.sync_copy(lengths_hbm, lens_v)
    lane = lax.iota(jnp.int32, _L)
    lens = lens_v[...]

    @pl.when(wid == 0)
    def _():
        nl_v[...] = lens // _RATE
        pltpu.sync_copy(nl_v, nl_hbm)

    zeros16 = jnp.zeros((_L,), jnp.float32)

    @pl.loop(0, _ZR)
    def _(r):
        for j in range(_D // _L):
            zbuf[r, pl.ds(j * _L, _L)] = zeros16

    pairb = wid % (_B // 2)
    q = wid // (_B // 2)
    vk, orow, row0 = [], [], []
    for i in range(_SLOTS):
        b = pairb if i % 2 == 0 else (_B - 1) - pairb
        j = q + 4 * (i // 2)
        nl = jnp.sum(jnp.where(lane == b, lens, 0)) // _RATE
        k = b * _JPB + j
        vk.append(jnp.clip(nl - j * _C, 0, _C))
        orow.append(k * _C)
        row0.append(b * _T + 2 * (j * _C) + 1)

    for i in range(_SLOTS):
        @pl.when(vk[i] == 0)
        def _(i=i):
            for z in range(_C // _ZR):
                pltpu.async_copy(
                    zbuf, out_hbm.at[pl.ds(orow[i] + z * _ZR, _ZR)], zsem)

    def gather_desc(i):
        s = i % _NBUF
        return pltpu.make_async_copy(
            padded_hbm.at[idxb[s]], gbuf[s], gsem[s])

    def out_desc(i):
        s = i % _NBUF
        return pltpu.make_async_copy(
            gbuf[s], out_hbm.at[pl.ds(orow[i], _C)], osem[s])

    def start_gather(i):
        s = i % _NBUF
        for j in range(_C // _L):
            idxb[s][pl.ds(j * _L, _L)] = row0[i] + 2 * (j * _L + lane)
        gather_desc(i).start()

    def finish_gather_start_out(i):
        gather_desc(i).wait()
        s = i % _NBUF

        @pl.when(vk[i] < _C)
        def _():
            @pl.loop(vk[i], _C)
            def _(r):
                for j in range(_D // _L):
                    gbuf[s][r, pl.ds(j * _L, _L)] = zeros16

        out_desc(i).start()

    for i in range(_SLOTS):
        if i >= _NBUF:
            @pl.when(vk[i - _NBUF] > 0)
            def _(i=i):
                out_desc(i - _NBUF).wait()

        @pl.when(vk[i] > 0)
        def _(i=i):
            start_gather(i)
        if i >= 1:
            @pl.when(vk[i - 1] > 0)
            def _(i=i):
                finish_gather_start_out(i - 1)

    @pl.when(vk[_SLOTS - 1] > 0)
    def _():
        finish_gather_start_out(_SLOTS - 1)

    for i in range(max(_SLOTS - _NBUF, 0), _SLOTS):
        @pl.when(vk[i] > 0)
        def _(i=i):
            out_desc(i).wait()

    for i in range(_SLOTS):
        @pl.when(vk[i] == 0)
        def _(i=i):
            for z in range(_C // _ZR):
                pltpu.make_async_copy(
                    zbuf, out_hbm.at[pl.ds(orow[i] + z * _ZR, _ZR)], zsem).wait()


def kernel(padded, lengths):
    padded2d = padded.reshape(_B * _T, _D)
    mesh = plsc.VectorSubcoreMesh(core_axis_name="c", subcore_axis_name="s")
    out2d, nl = pl.kernel(
        _sc_body,
        out_type=(
            jax.ShapeDtypeStruct((_B * _TO, _D), jnp.float32),
            jax.ShapeDtypeStruct((_B,), jnp.int32),
        ),
        mesh=mesh,
        compiler_params=pltpu.CompilerParams(needs_layout_passes=False),
        scratch_types=(
            *([pltpu.VMEM((_C,), jnp.int32)] * _NBUF),
            *([pltpu.VMEM((_C, _D), jnp.float32)] * _NBUF),
            pltpu.VMEM((_ZR, _D), jnp.float32),
            pltpu.VMEM((_L,), jnp.int32),
            pltpu.VMEM((_L,), jnp.int32),
            *([pltpu.SemaphoreType.DMA] * _NBUF),
            *([pltpu.SemaphoreType.DMA] * _NBUF),
            pltpu.SemaphoreType.DMA,
        ),
    )(padded2d, lengths)
    return out2d.reshape(_B, _TO, _D), nl


# speedup vs baseline: 1.0594x; 1.0079x over previous
import jax
import jax.numpy as jnp
from jax import lax
from jax.experimental import pallas as pl
from jax.experimental.pallas import tpu as pltpu
from jax.experimental.pallas import tpu_sc as plsc

_RATE = 2
_B, _T, _D = 16, 4096, 256
_TO = _T // _RATE
_C = 128
_NWORK = 32
_NCHG = (_B * _TO) // _C
_SLOTS = _NCHG // _NWORK
_JPB = _TO // _C
_L = 16
_NBUF = 3
_ZR = 64
_PERM = 173


def _sc_body(padded_hbm, lengths_hbm, out_hbm, nl_hbm,
             idx0, idx1, idx2, gbuf0, gbuf1, gbuf2, zbuf, lens_v, nl_v,
             gsem0, gsem1, gsem2, osem0, osem1, osem2, zsem):
    idxb = (idx0, idx1, idx2)
    gbuf = (gbuf0, gbuf1, gbuf2)
    gsem = (gsem0, gsem1, gsem2)
    osem = (osem0, osem1, osem2)

    wid = lax.axis_index("s") * 2 + lax.axis_index("c")

    pltpu.sync_copy(lengths_hbm, lens_v)
    lane = lax.iota(jnp.int32, _L)
    lens = lens_v[...]

    zeros16 = jnp.zeros((_L,), jnp.float32)

    pairb = wid % (_B // 2)
    q = wid // (_B // 2)
    vk, orow, row0 = [], [], []
    for i in range(_SLOTS):
        b = pairb if i % 2 == 0 else (_B - 1) - pairb
        j = q + 4 * (i // 2)
        nl = jnp.sum(jnp.where(lane == b, lens, 0)) // _RATE
        k = b * _JPB + j
        vk.append(jnp.clip(nl - j * _C, 0, _C))
        orow.append(k * _C)
        row0.append(b * _T + 2 * (j * _C) + 1)

    def gather_desc(i):
        s = i % _NBUF
        return pltpu.make_async_copy(
            padded_hbm.at[idxb[s]], gbuf[s], gsem[s])

    def out_desc(i):
        s = i % _NBUF
        return pltpu.make_async_copy(
            gbuf[s], out_hbm.at[pl.ds(orow[i], _C)], osem[s])

    def start_gather(i):
        s = i % _NBUF
        for j in range(_C // _L):
            idxb[s][pl.ds(j * _L, _L)] = row0[i] + 2 * (j * _L + lane)
        gather_desc(i).start()

    def finish_gather_start_out(i):
        gather_desc(i).wait()
        s = i % _NBUF

        @pl.when(vk[i] < _C)
        def _():
            @pl.loop(vk[i], _C)
            def _(r):
                for j in range(_D // _L):
                    gbuf[s][r, pl.ds(j * _L, _L)] = zeros16

        out_desc(i).start()

    for i in range(_NBUF):
        @pl.when(vk[i] > 0)
        def _(i=i):
            start_gather(i)

    @pl.when(wid == 0)
    def _():
        nl_v[...] = lens // _RATE
        pltpu.sync_copy(nl_v, nl_hbm)

    @pl.loop(0, _ZR)
    def _(r):
        for j in range(_D // _L):
            zbuf[r, pl.ds(j * _L, _L)] = zeros16

    for i in range(_SLOTS):
        @pl.when(vk[i] == 0)
        def _(i=i):
            for z in range(_C // _ZR):
                pltpu.async_copy(
                    zbuf, out_hbm.at[pl.ds(orow[i] + z * _ZR, _ZR)], zsem)

    for i in range(_SLOTS):
        if i >= _NBUF:
            @pl.when(vk[i - _NBUF] > 0)
            def _(i=i):
                out_desc(i - _NBUF).wait()

            @pl.when(vk[i] > 0)
            def _(i=i):
                start_gather(i)
        if i >= 1:
            @pl.when(vk[i - 1] > 0)
            def _(i=i):
                finish_gather_start_out(i - 1)

    @pl.when(vk[_SLOTS - 1] > 0)
    def _():
        finish_gather_start_out(_SLOTS - 1)

    for i in range(max(_SLOTS - _NBUF, 0), _SLOTS):
        @pl.when(vk[i] > 0)
        def _(i=i):
            out_desc(i).wait()

    for i in range(_SLOTS):
        @pl.when(vk[i] == 0)
        def _(i=i):
            for z in range(_C // _ZR):
                pltpu.make_async_copy(
                    zbuf, out_hbm.at[pl.ds(orow[i] + z * _ZR, _ZR)], zsem).wait()


def kernel(padded, lengths):
    padded2d = padded.reshape(_B * _T, _D)
    mesh = plsc.VectorSubcoreMesh(core_axis_name="c", subcore_axis_name="s")
    out2d, nl = pl.kernel(
        _sc_body,
        out_type=(
            jax.ShapeDtypeStruct((_B * _TO, _D), jnp.float32),
            jax.ShapeDtypeStruct((_B,), jnp.int32),
        ),
        mesh=mesh,
        compiler_params=pltpu.CompilerParams(needs_layout_passes=False),
        scratch_types=(
            *([pltpu.VMEM((_C,), jnp.int32)] * _NBUF),
            *([pltpu.VMEM((_C, _D), jnp.float32)] * _NBUF),
            pltpu.VMEM((_ZR, _D), jnp.float32),
            pltpu.VMEM((_L,), jnp.int32),
            pltpu.VMEM((_L,), jnp.int32),
            *([pltpu.SemaphoreType.DMA] * _NBUF),
            *([pltpu.SemaphoreType.DMA] * _NBUF),
            pltpu.SemaphoreType.DMA,
        ),
    )(padded2d, lengths)
    return out2d.reshape(_B, _TO, _D), nl


# speedup vs baseline: 1.0625x; 1.0029x over previous
import jax
import jax.numpy as jnp
from jax import lax
from jax.experimental import pallas as pl
from jax.experimental.pallas import tpu as pltpu
from jax.experimental.pallas import tpu_sc as plsc

_RATE = 2
_B, _T, _D = 16, 4096, 256
_TO = _T // _RATE
_C = 128
_NWORK = 32
_NCHG = (_B * _TO) // _C
_SLOTS = _NCHG // _NWORK
_JPB = _TO // _C
_L = 16
_NBUF = 3
_ZR = 64
_PERM = 173


def _sc_body(padded_hbm, lengths_hbm, out_hbm, nl_hbm,
             idxall, gbuf0, gbuf1, gbuf2, zbuf, lens_v, nl_v,
             gsem0, gsem1, gsem2, osem0, osem1, osem2, zsem):
    gbuf = (gbuf0, gbuf1, gbuf2)
    gsem = (gsem0, gsem1, gsem2)
    osem = (osem0, osem1, osem2)

    wid = lax.axis_index("s") * 2 + lax.axis_index("c")

    pltpu.sync_copy(lengths_hbm, lens_v)
    lane = lax.iota(jnp.int32, _L)
    lens = lens_v[...]

    zeros16 = jnp.zeros((_L,), jnp.float32)

    pairb = wid % (_B // 2)
    q = wid // (_B // 2)
    vk, orow, row0 = [], [], []
    for i in range(_SLOTS):
        b = pairb if i % 2 == 0 else (_B - 1) - pairb
        j = q + 4 * (i // 2)
        nl = jnp.sum(jnp.where(lane == b, lens, 0)) // _RATE
        k = b * _JPB + j
        vk.append(jnp.clip(nl - j * _C, 0, _C))
        orow.append(k * _C)
        row0.append(b * _T + 2 * (j * _C) + 1)

    @pl.loop(0, _SLOTS * _C // _L)
    def _(n):
        i = n // (_C // _L)
        jj = n % (_C // _L)
        bi = jnp.where(i % 2 == 0, pairb, (_B - 1) - pairb)
        jpos = q + 4 * (i // 2)
        r0 = bi * _T + 2 * (jpos * _C) + 1
        idxall[pl.ds(n * _L, _L)] = r0 + 2 * (jj * _L + lane)

    def gather_desc(i):
        s = i % _NBUF
        return pltpu.make_async_copy(
            padded_hbm.at[idxall.at[pl.ds(i * _C, _C)]], gbuf[s], gsem[s])

    def out_desc(i):
        s = i % _NBUF
        return pltpu.make_async_copy(
            gbuf[s], out_hbm.at[pl.ds(orow[i], _C)], osem[s])

    def start_gather(i):
        gather_desc(i).start()

    def finish_gather_start_out(i):
        gather_desc(i).wait()
        s = i % _NBUF

        @pl.when(vk[i] < _C)
        def _():
            @pl.loop(vk[i], _C)
            def _(r):
                for j in range(_D // _L):
                    gbuf[s][r, pl.ds(j * _L, _L)] = zeros16

        out_desc(i).start()

    for i in range(_NBUF):
        @pl.when(vk[i] > 0)
        def _(i=i):
            start_gather(i)

    @pl.when(wid == 0)
    def _():
        nl_v[...] = lens // _RATE
        pltpu.sync_copy(nl_v, nl_hbm)

    @pl.loop(0, _ZR)
    def _(r):
        for j in range(_D // _L):
            zbuf[r, pl.ds(j * _L, _L)] = zeros16

    def slot_vk_orow(i):
        bi = jnp.where(i % 2 == 0, pairb, (_B - 1) - pairb)
        jpos = q + 4 * (i // 2)
        nli = jnp.sum(jnp.where(lane == bi, lens, 0)) // _RATE
        return (jnp.clip(nli - jpos * _C, 0, _C),
                (bi * _JPB + jpos) * _C)

    @pl.loop(0, _SLOTS)
    def _(i):
        vki, orowi = slot_vk_orow(i)

        @pl.when(vki == 0)
        def _():
            for z in range(_C // _ZR):
                pltpu.async_copy(
                    zbuf, out_hbm.at[pl.ds(orowi + z * _ZR, _ZR)], zsem)

    for i in range(_SLOTS):
        if i >= _NBUF:
            @pl.when(vk[i - _NBUF] > 0)
            def _(i=i):
                out_desc(i - _NBUF).wait()

            @pl.when(vk[i] > 0)
            def _(i=i):
                start_gather(i)
        if i >= 1:
            @pl.when(vk[i - 1] > 0)
            def _(i=i):
                finish_gather_start_out(i - 1)

    @pl.when(vk[_SLOTS - 1] > 0)
    def _():
        finish_gather_start_out(_SLOTS - 1)

    for i in range(max(_SLOTS - _NBUF, 0), _SLOTS):
        @pl.when(vk[i] > 0)
        def _(i=i):
            out_desc(i).wait()

    @pl.loop(0, _SLOTS)
    def _(i):
        vki, orowi = slot_vk_orow(i)

        @pl.when(vki == 0)
        def _():
            for z in range(_C // _ZR):
                pltpu.make_async_copy(
                    zbuf, out_hbm.at[pl.ds(orowi + z * _ZR, _ZR)], zsem).wait()


def kernel(padded, lengths):
    padded2d = padded.reshape(_B * _T, _D)
    mesh = plsc.VectorSubcoreMesh(core_axis_name="c", subcore_axis_name="s")
    out2d, nl = pl.kernel(
        _sc_body,
        out_type=(
            jax.ShapeDtypeStruct((_B * _TO, _D), jnp.float32),
            jax.ShapeDtypeStruct((_B,), jnp.int32),
        ),
        mesh=mesh,
        compiler_params=pltpu.CompilerParams(needs_layout_passes=False),
        scratch_types=(
            pltpu.VMEM((_SLOTS * _C,), jnp.int32),
            *([pltpu.VMEM((_C, _D), jnp.float32)] * _NBUF),
            pltpu.VMEM((_ZR, _D), jnp.float32),
            pltpu.VMEM((_L,), jnp.int32),
            pltpu.VMEM((_L,), jnp.int32),
            *([pltpu.SemaphoreType.DMA] * _NBUF),
            *([pltpu.SemaphoreType.DMA] * _NBUF),
            pltpu.SemaphoreType.DMA,
        ),
    )(padded2d, lengths)
    return out2d.reshape(_B, _TO, _D), nl


# speedup vs baseline: 1.0638x; 1.0013x over previous
import jax
import jax.numpy as jnp
from jax import lax
from jax.experimental import pallas as pl
from jax.experimental.pallas import tpu as pltpu
from jax.experimental.pallas import tpu_sc as plsc

_RATE = 2
_B, _T, _D = 16, 4096, 256
_TO = _T // _RATE
_C = 128
_NWORK = 32
_NCHG = (_B * _TO) // _C
_SLOTS = _NCHG // _NWORK
_JPB = _TO // _C
_L = 16
_NBUF = 3
_ZR = 64


def _sc_body(padded_hbm, lengths_hbm, out_hbm, nl_hbm,
             idxall, gbuf0, gbuf1, gbuf2, zbuf, lens_v, nl_v,
             gsem0, gsem1, gsem2, osem0, osem1, osem2, zsem):
    gbuf = (gbuf0, gbuf1, gbuf2)
    gsem = (gsem0, gsem1, gsem2)
    osem = (osem0, osem1, osem2)

    wid = lax.axis_index("s") * 2 + lax.axis_index("c")

    pltpu.sync_copy(lengths_hbm, lens_v)
    lane = lax.iota(jnp.int32, _L)
    lens = lens_v[...]

    zeros16 = jnp.zeros((_L,), jnp.float32)

    pairb = wid % (_B // 2)
    q = wid // (_B // 2)
    vk, orow = [], []
    for i in range(_SLOTS):
        b = pairb if i % 2 == 0 else (_B - 1) - pairb
        j = q + 4 * (i // 2)
        nl = jnp.sum(jnp.where(lane == b, lens, 0)) // _RATE
        vk.append(jnp.clip(nl - j * _C, 0, _C))
        orow.append((b * _JPB + j) * _C)

    @pl.loop(0, _SLOTS * _C // _L)
    def _(n):
        i = n // (_C // _L)
        jj = n % (_C // _L)
        bi = jnp.where(i % 2 == 0, pairb, (_B - 1) - pairb)
        jpos = q + 4 * (i // 2)
        r0 = bi * _T + 2 * (jpos * _C) + 1
        idxall[pl.ds(n * _L, _L)] = r0 + 2 * (jj * _L + lane)

    def gather_desc(i):
        s = i % _NBUF
        return pltpu.make_async_copy(
            padded_hbm.at[idxall.at[pl.ds(i * _C, _C)]], gbuf[s], gsem[s])

    def out_desc(i):
        s = i % _NBUF
        return pltpu.make_async_copy(
            gbuf[s], out_hbm.at[pl.ds(orow[i], _C)], osem[s])

    def start_gather(i):
        gather_desc(i).start()

    def finish_gather_start_out(i):
        gather_desc(i).wait()
        s = i % _NBUF

        @pl.when(vk[i] < _C)
        def _():
            @pl.loop(vk[i], _C)
            def _(r):
                for j in range(_D // _L):
                    gbuf[s][r, pl.ds(j * _L, _L)] = zeros16

        out_desc(i).start()

    for i in range(_NBUF):
        @pl.when(vk[i] > 0)
        def _(i=i):
            start_gather(i)

    @pl.when(wid == 0)
    def _():
        nl_v[...] = lens // _RATE
        pltpu.sync_copy(nl_v, nl_hbm)

    @pl.loop(0, _ZR)
    def _(r):
        for j in range(_D // _L):
            zbuf[r, pl.ds(j * _L, _L)] = zeros16

    def slot_vk_orow(i):
        bi = jnp.where(i % 2 == 0, pairb, (_B - 1) - pairb)
        jpos = q + 4 * (i // 2)
        nli = jnp.sum(jnp.where(lane == bi, lens, 0)) // _RATE
        return (jnp.clip(nli - jpos * _C, 0, _C),
                (bi * _JPB + jpos) * _C)

    @pl.loop(0, _SLOTS)
    def _(i):
        vki, orowi = slot_vk_orow(i)

        @pl.when(vki == 0)
        def _():
            for z in range(_C // _ZR):
                pltpu.async_copy(
                    zbuf, out_hbm.at[pl.ds(orowi + z * _ZR, _ZR)], zsem)

    for i in range(_SLOTS):
        if i >= _NBUF:
            @pl.when(vk[i - _NBUF] > 0)
            def _(i=i):
                out_desc(i - _NBUF).wait()

            @pl.when(vk[i] > 0)
            def _(i=i):
                start_gather(i)
        if i >= 1:
            @pl.when(vk[i - 1] > 0)
            def _(i=i):
                finish_gather_start_out(i - 1)

    @pl.when(vk[_SLOTS - 1] > 0)
    def _():
        finish_gather_start_out(_SLOTS - 1)

    for i in range(max(_SLOTS - _NBUF, 0), _SLOTS):
        @pl.when(vk[i] > 0)
        def _(i=i):
            out_desc(i).wait()

    @pl.loop(0, _SLOTS)
    def _(i):
        vki, orowi = slot_vk_orow(i)

        @pl.when(vki == 0)
        def _():
            for z in range(_C // _ZR):
                pltpu.make_async_copy(
                    zbuf, out_hbm.at[pl.ds(orowi + z * _ZR, _ZR)], zsem).wait()


def kernel(padded, lengths):
    padded2d = padded.reshape(_B * _T, _D)
    mesh = plsc.VectorSubcoreMesh(core_axis_name="c", subcore_axis_name="s")
    out2d, nl = pl.kernel(
        _sc_body,
        out_type=(
            jax.ShapeDtypeStruct((_B * _TO, _D), jnp.float32),
            jax.ShapeDtypeStruct((_B,), jnp.int32),
        ),
        mesh=mesh,
        compiler_params=pltpu.CompilerParams(needs_layout_passes=False),
        scratch_types=(
            pltpu.VMEM((_SLOTS * _C,), jnp.int32),
            *([pltpu.VMEM((_C, _D), jnp.float32)] * _NBUF),
            pltpu.VMEM((_ZR, _D), jnp.float32),
            pltpu.VMEM((_L,), jnp.int32),
            pltpu.VMEM((_L,), jnp.int32),
            *([pltpu.SemaphoreType.DMA] * _NBUF),
            *([pltpu.SemaphoreType.DMA] * _NBUF),
            pltpu.SemaphoreType.DMA,
        ),
    )(padded2d, lengths)
    return out2d.reshape(_B, _TO, _D), nl
